# baseline (speedup 1.0000x reference)
import functools

import jax
import jax.numpy as jnp
from jax import lax
from jax.experimental import pallas as pl
from jax.experimental.pallas import tpu as pltpu
from jax.experimental.pallas import tpu_sc as plsc

EMB = 128

_MESH = plsc.VectorSubcoreMesh(core_axis_name="c", subcore_axis_name="s")


@functools.partial(
    pl.kernel,
    mesh=_MESH,
    out_type=jax.ShapeDtypeStruct((1, EMB), jnp.float32),
    scratch_types=[
        pltpu.VMEM((1,), jnp.int32),
        pltpu.VMEM((1, EMB), jnp.float32),
        pltpu.SemaphoreType.DMA,
    ],
)
def _sc_lookup(idx_hbm, table_hbm, out_hbm, idx_v, row_v, sem):
    is_lead = (lax.axis_index("c") == 0) & (lax.axis_index("s") == 0)

    @pl.when(is_lead)
    def _():
        pltpu.sync_copy(idx_hbm, idx_v)
        pltpu.async_copy(table_hbm.at[idx_v], row_v, sem).wait()
        pltpu.sync_copy(row_v, out_hbm)


def kernel(token, table):
    idx = jnp.reshape(jnp.asarray(token, jnp.int32), (1,))
    out = _sc_lookup(idx, table)
    return jnp.reshape(out, (EMB,))


# speedup vs baseline: 1.0806x; 1.0806x over previous
import functools

import jax
import jax.numpy as jnp
from jax import lax
from jax.experimental import pallas as pl
from jax.experimental.pallas import tpu as pltpu
from jax.experimental.pallas import tpu_sc as plsc

EMB = 128

_MESH = plsc.VectorSubcoreMesh(
    core_axis_name="c", subcore_axis_name="s", num_cores=1, num_subcores=1
)


@functools.partial(
    pl.kernel,
    mesh=_MESH,
    out_type=jax.ShapeDtypeStruct((1, EMB), jnp.float32),
    scratch_types=[
        pltpu.VMEM((1,), jnp.int32),
        pltpu.VMEM((1, EMB), jnp.float32),
        pltpu.SemaphoreType.DMA,
    ],
)
def _sc_lookup(idx_hbm, table_hbm, out_hbm, idx_v, row_v, sem):
    pltpu.sync_copy(idx_hbm, idx_v)
    pltpu.async_copy(table_hbm.at[idx_v], row_v, sem).wait()
    pltpu.sync_copy(row_v, out_hbm)


def kernel(token, table):
    idx = jnp.reshape(jnp.asarray(token, jnp.int32), (1,))
    out = _sc_lookup(idx, table)
    return jnp.reshape(out, (EMB,))


# speedup vs baseline: 1.1875x; 1.0989x over previous
import functools

import jax
import jax.numpy as jnp
from jax import lax
from jax.experimental import pallas as pl
from jax.experimental.pallas import tpu as pltpu
from jax.experimental.pallas import tpu_sc as plsc

EMB = 128

_MESH = plsc.ScalarSubcoreMesh(axis_name="c", num_cores=1)


@functools.partial(
    pl.kernel,
    mesh=_MESH,
    out_type=jax.ShapeDtypeStruct((1, EMB), jnp.float32),
    scratch_types=[
        pltpu.SMEM((1,), jnp.int32),
    ],
)
def _sc_lookup(idx_hbm, table_hbm, out_hbm, idx_s):
    pltpu.sync_copy(idx_hbm, idx_s)
    tok = idx_s[0]
    pltpu.sync_copy(table_hbm.at[pl.ds(tok, 1)], out_hbm)


def kernel(token, table):
    idx = jnp.reshape(jnp.asarray(token, jnp.int32), (1,))
    out = _sc_lookup(idx, table)
    return jnp.reshape(out, (EMB,))


# speedup vs baseline: 1.1887x; 1.0010x over previous
import functools

import jax
import jax.numpy as jnp
from jax import lax
from jax.experimental import pallas as pl
from jax.experimental.pallas import tpu as pltpu
from jax.experimental.pallas import tpu_sc as plsc

EMB = 128

_MESH = plsc.ScalarSubcoreMesh(axis_name="c", num_cores=1)


@functools.partial(
    pl.kernel,
    mesh=_MESH,
    out_type=jax.ShapeDtypeStruct((EMB,), jnp.float32),
    scratch_types=[
        pltpu.SMEM((1,), jnp.int32),
    ],
)
def _sc_lookup(idx_hbm, table_hbm, out_hbm, idx_s):
    pltpu.sync_copy(idx_hbm, idx_s)
    tok = idx_s[0]
    pltpu.sync_copy(table_hbm.at[tok], out_hbm)


def kernel(token, table):
    idx = jnp.reshape(jnp.asarray(token, jnp.int32), (1,))
    return _sc_lookup(idx, table)


# speedup vs baseline: 1.2009x; 1.0103x over previous
import functools

import jax
import jax.numpy as jnp
from jax.experimental import pallas as pl
from jax.experimental.pallas import tpu as pltpu
from jax.experimental.pallas import tpu_sc as plsc

EMB = 128

_MESH = plsc.ScalarSubcoreMesh(axis_name="c", num_cores=1)


@functools.partial(
    pl.kernel,
    mesh=_MESH,
    out_type=jax.ShapeDtypeStruct((EMB,), jnp.float32),
    scratch_types=[
        pltpu.SMEM((1,), jnp.int32),
    ],
)
def _sc_lookup(idx_hbm, table_hbm, out_hbm, idx_s):
    pltpu.sync_copy(idx_hbm, idx_s)
    tok = idx_s[0]
    pltpu.sync_copy(table_hbm.at[tok], out_hbm)


def kernel(token, table):
    idx = jnp.reshape(jnp.asarray(token, jnp.int32), (1,))
    return _sc_lookup(idx, table)
